# baseline (speedup 1.0000x reference)
import functools

import jax
import jax.numpy as jnp
import numpy as np
from jax import lax
from jax.experimental import pallas as pl
from jax.experimental.pallas import tpu as pltpu
from jax.experimental.pallas import tpu_sc as plsc

N = 10000
E = 320000
D = 128
H = 128
MID = 32
O = 64
B = 16

NC = 2
NS = 16
NW = NC * NS
EPT = E // NW
CH = 80
NCHUNK = EPT // CH
BC = 32
N_PAD = 10112
STRIPE = N_PAD // NS


def _sc_aggregate(h, idx3, zeros_hbm):
  mesh = plsc.VectorSubcoreMesh(core_axis_name="c", subcore_axis_name="s")

  @functools.partial(
      pl.kernel,
      out_type=jax.ShapeDtypeStruct((NC, N_PAD, D), jnp.float32),
      mesh=mesh,
      scratch_types=[
          pltpu.VMEM((BC, 2, CH), jnp.int32),
          pltpu.VMEM((BC, 2, CH), jnp.int32),
          pltpu.VMEM((CH, D), jnp.float32),
          pltpu.VMEM((CH, D), jnp.float32),
          pltpu.VMEM((CH, D), jnp.float32),
          pltpu.VMEM_SHARED((N_PAD, D), jnp.float32),
          pltpu.SemaphoreType.DMA,
          pltpu.SemaphoreType.DMA,
          pltpu.SemaphoreType.DMA,
          pltpu.SemaphoreType.DMA,
      ],
  )
  def agg_kernel(h_hbm, idx_hbm, z_hbm, out_hbm, idx_v0, idx_v1,
                 rows_a, rows_b, rows_c, acc_s,
                 sem_a, sem_b, sem_c, sem_i):
    c = lax.axis_index("c")
    s = lax.axis_index("s")
    w = c * NS + s

    pltpu.sync_copy(z_hbm.at[pl.ds(s * STRIPE, STRIPE)],
                    acc_s.at[pl.ds(s * STRIPE, STRIPE)])
    plsc.subcore_barrier()

    def _wait(buf, sem):
      pltpu.make_async_copy(h_hbm.at[pl.ds(0, CH)], buf, sem).wait()

    def _gather(iv, j, buf, sem):
      pltpu.async_copy(h_hbm.at[iv.at[j, 1]], buf, sem)

    def _scat(iv, buf, j):
      pltpu.sync_copy(buf, acc_s.at[iv.at[j, 0]], add=True)

    bufs = ((rows_a, sem_a), (rows_b, sem_b), (rows_c, sem_c))
    K = len(bufs)
    F = K - 1

    def _run_block(iv, cnt):
      for t in range(F):
        _gather(iv, t, *bufs[t])
      groups, rem = divmod(cnt - F, K)

      def _group(k, carry):
        j = K * k
        for t in range(K):
          buf, sem = bufs[t]
          _wait(buf, sem)
          _gather(iv, j + t + F, *bufs[(t + F) % K])
          _scat(iv, buf, j + t)
        return carry

      lax.fori_loop(0, groups, _group, 0)
      base = K * groups
      for t in range(rem):
        buf, sem = bufs[t % K]
        _wait(buf, sem)
        _gather(iv, base + t + F, *bufs[(t + F) % K])
        _scat(iv, buf, base + t)
      for t in range(rem, rem + F):
        buf, sem = bufs[t % K]
        _wait(buf, sem)
        _scat(iv, buf, cnt - F + (t - rem))

    blocks = ((0, 31), (31, 31), (62, 31), (93, 32))
    ibufs = (idx_v0, idx_v1)
    pltpu.sync_copy(idx_hbm.at[w, pl.ds(0, BC)], ibufs[0])
    for b, (off, cnt) in enumerate(blocks):
      if b + 1 < len(blocks):
        noff = blocks[b + 1][0]
        pltpu.async_copy(idx_hbm.at[w, pl.ds(noff, BC)],
                         ibufs[(b + 1) % 2], sem_i)
      _run_block(ibufs[b % 2], cnt)
      if b + 1 < len(blocks):
        pltpu.make_async_copy(idx_hbm.at[w, pl.ds(0, BC)],
                              ibufs[(b + 1) % 2], sem_i).wait()
    plsc.subcore_barrier()

    pltpu.sync_copy(acc_s.at[pl.ds(s * STRIPE, STRIPE)],
                    out_hbm.at[c, pl.ds(s * STRIPE, STRIPE)])

  return agg_kernel(h, idx3, zeros_hbm)


def _mlp_bn(u, W1_ref, b1_ref, bng_ref, bnb_ref, W2_ref, b2_ref, obng_ref,
            obnb_ref):
  t = jnp.dot(u, W1_ref[...], preferred_element_type=jnp.float32,
              precision=lax.Precision.DEFAULT) + b1_ref[...]
  m = jnp.mean(t, axis=0, keepdims=True)
  v = jnp.mean((t - m) ** 2, axis=0, keepdims=True)
  t = (t - m) * lax.rsqrt(v + 1e-5) * bng_ref[...] + bnb_ref[...]
  t = jnp.maximum(t, 0.0)
  t = jnp.dot(t, W2_ref[...], preferred_element_type=jnp.float32,
              precision=lax.Precision.DEFAULT) + b2_ref[...]
  m2 = jnp.mean(t, axis=0, keepdims=True)
  v2 = jnp.mean((t - m2) ** 2, axis=0, keepdims=True)
  t = (t - m2) * lax.rsqrt(v2 + 1e-5) * obng_ref[...] + obnb_ref[...]
  return jnp.maximum(t, 0.0)


def _conv_tc(x, parts, W1, b1, bng, bnb, W2, b2, obng, obnb):
  def body(x_ref, parts_ref, W1_ref, b1_ref, bng_ref, bnb_ref, W2_ref, b2_ref,
           obng_ref, obnb_ref, out_ref):
    u = x_ref[...] + parts_ref[0, :N, :] + parts_ref[1, :N, :]
    out_ref[...] = _mlp_bn(u, W1_ref, b1_ref, bng_ref, bnb_ref, W2_ref,
                           b2_ref, obng_ref, obnb_ref)

  return pl.pallas_call(
      body,
      out_shape=jax.ShapeDtypeStruct((N, D), jnp.float32),
  )(x, parts, W1, b1, bng, bnb, W2, b2, obng, obnb)


def _head_tc(x, h1, parts, W1, b1, bng, bnb, W2, b2, obng, obnb, bidx,
             p0_W, p0_b, p2_W, p2_b, out_W, out_b):
  def body(x_ref, h1_ref, parts_ref, W1_ref, b1_ref, bng_ref, bnb_ref,
           W2_ref, b2_ref, obng_ref, obnb_ref, bidx_ref, p0W_ref, p0b_ref,
           p2W_ref, p2b_ref, outW_ref, outb_ref, out_ref):
    u = h1_ref[...] + parts_ref[0, :N, :] + parts_ref[1, :N, :]
    h2 = _mlp_bn(u, W1_ref, b1_ref, bng_ref, bnb_ref, W2_ref, b2_ref,
                 obng_ref, obnb_ref)
    seg = lax.broadcasted_iota(jnp.int32, (B, N), 0)
    onehot = (seg == bidx_ref[...]).astype(jnp.float32)
    pool0 = jnp.dot(onehot, x_ref[...], preferred_element_type=jnp.float32,
                    precision=lax.Precision.DEFAULT)
    pool2 = jnp.dot(onehot, h2, preferred_element_type=jnp.float32,
                    precision=lax.Precision.DEFAULT)
    oh = (jnp.dot(pool0, p0W_ref[...], preferred_element_type=jnp.float32,
                  precision=lax.Precision.DEFAULT) + p0b_ref[...]
          + jnp.dot(pool2, p2W_ref[...], preferred_element_type=jnp.float32,
                    precision=lax.Precision.DEFAULT) + p2b_ref[...])
    oh = jnp.maximum(oh, 0.0)
    out_ref[...] = jnp.dot(oh, outW_ref[...],
                           preferred_element_type=jnp.float32,
                           precision=lax.Precision.DEFAULT) + outb_ref[...]

  return pl.pallas_call(
      body,
      out_shape=jax.ShapeDtypeStruct((B, O), jnp.float32),
  )(x, h1, parts, W1, b1, bng, bnb, W2, b2, obng, obnb, bidx,
    p0_W, p0_b, p2_W, p2_b, out_W, out_b)


def kernel(x, A, batch_index,
           g0_W1, g0_b1, g0_bng, g0_bnb, g0_W2, g0_b2, g0_obng, g0_obnb,
           g1_W1, g1_b1, g1_bng, g1_bnb, g1_W2, g1_b2, g1_obng, g1_obnb,
           p0_W, p0_b, p2_W, p2_b, out_W, out_b):
  idx3 = A.reshape(2, NW, NCHUNK, CH).transpose(1, 2, 0, 3)
  bidx = batch_index.reshape(1, N)

  r2 = lambda a: a.reshape(1, -1)
  zeros_hbm = jnp.asarray(np.zeros((N_PAD, D), np.float32))

  parts0 = _sc_aggregate(x, idx3, zeros_hbm)
  h1 = _conv_tc(x, parts0, g0_W1, r2(g0_b1), r2(g0_bng), r2(g0_bnb),
                g0_W2, r2(g0_b2), r2(g0_obng), r2(g0_obnb))
  parts1 = _sc_aggregate(h1, idx3, zeros_hbm)
  return _head_tc(x, h1, parts1, g1_W1, r2(g1_b1), r2(g1_bng), r2(g1_bnb),
                  g1_W2, r2(g1_b2), r2(g1_obng), r2(g1_obnb), bidx,
                  p0_W, r2(p0_b), p2_W, r2(p2_b), out_W, r2(out_b))


# speedup vs baseline: 1.0272x; 1.0272x over previous
import functools

import jax
import jax.numpy as jnp
import numpy as np
from jax import lax
from jax.experimental import pallas as pl
from jax.experimental.pallas import tpu as pltpu
from jax.experimental.pallas import tpu_sc as plsc

N = 10000
E = 320000
D = 128
H = 128
MID = 32
O = 64
B = 16

NC = 2
NS = 16
NW = NC * NS
EPT = E // NW
CH = 80
NCHUNK = EPT // CH
BC = 32
N_PAD = 10112
STRIPE = N_PAD // NS


def _sc_aggregate(h, idx3, zeros_hbm):
  mesh = plsc.VectorSubcoreMesh(core_axis_name="c", subcore_axis_name="s")

  @functools.partial(
      pl.kernel,
      out_type=jax.ShapeDtypeStruct((NC, N_PAD, D), jnp.float32),
      mesh=mesh,
      scratch_types=[
          pltpu.VMEM((BC, 2, CH), jnp.int32),
          pltpu.VMEM((CH, D), jnp.float32),
          pltpu.VMEM((CH, D), jnp.float32),
          pltpu.VMEM((CH, D), jnp.float32),
          pltpu.VMEM((CH, D), jnp.float32),
          pltpu.VMEM_SHARED((N_PAD, D), jnp.float32),
          pltpu.SemaphoreType.DMA,
          pltpu.SemaphoreType.DMA,
          pltpu.SemaphoreType.DMA,
          pltpu.SemaphoreType.DMA,
      ],
  )
  def agg_kernel(h_hbm, idx_hbm, z_hbm, out_hbm, idx_v,
                 rows_a, rows_b, rows_c, rows_e, acc_s,
                 sem_a, sem_b, sem_c, sem_e):
    c = lax.axis_index("c")
    s = lax.axis_index("s")
    w = c * NS + s

    pltpu.sync_copy(z_hbm.at[pl.ds(s * STRIPE, STRIPE)],
                    acc_s.at[pl.ds(s * STRIPE, STRIPE)])
    plsc.subcore_barrier()

    def _wait(buf, sem):
      pltpu.make_async_copy(h_hbm.at[pl.ds(0, CH)], buf, sem).wait()

    def _gather(iv, j, buf, sem):
      pltpu.async_copy(h_hbm.at[iv.at[j, 1]], buf, sem)

    def _scat(iv, buf, j):
      pltpu.sync_copy(buf, acc_s.at[iv.at[j, 0]], add=True)

    bufs = ((rows_a, sem_a), (rows_b, sem_b), (rows_c, sem_c),
            (rows_e, sem_e))
    K = len(bufs)
    F = K - 1

    def _run_block(iv, cnt):
      for t in range(F):
        _gather(iv, t, *bufs[t])
      groups, rem = divmod(cnt - F, K)

      def _group(k, carry):
        j = K * k
        for t in range(K):
          buf, sem = bufs[t]
          _wait(buf, sem)
          _gather(iv, j + t + F, *bufs[(t + F) % K])
          _scat(iv, buf, j + t)
        return carry

      lax.fori_loop(0, groups, _group, 0)
      base = K * groups
      for t in range(rem):
        buf, sem = bufs[t % K]
        _wait(buf, sem)
        _gather(iv, base + t + F, *bufs[(t + F) % K])
        _scat(iv, buf, base + t)
      for t in range(rem, rem + F):
        buf, sem = bufs[t % K]
        _wait(buf, sem)
        _scat(iv, buf, cnt - F + (t - rem))

    for off, cnt in ((0, 31), (31, 31), (62, 31), (93, 32)):
      pltpu.sync_copy(idx_hbm.at[w, pl.ds(off, BC)], idx_v)
      _run_block(idx_v, cnt)
    plsc.subcore_barrier()

    pltpu.sync_copy(acc_s.at[pl.ds(s * STRIPE, STRIPE)],
                    out_hbm.at[c, pl.ds(s * STRIPE, STRIPE)])

  return agg_kernel(h, idx3, zeros_hbm)


def _mlp_bn(u, W1_ref, b1_ref, bng_ref, bnb_ref, W2_ref, b2_ref, obng_ref,
            obnb_ref):
  t = jnp.dot(u, W1_ref[...], preferred_element_type=jnp.float32,
              precision=lax.Precision.DEFAULT) + b1_ref[...]
  m = jnp.mean(t, axis=0, keepdims=True)
  v = jnp.mean((t - m) ** 2, axis=0, keepdims=True)
  t = (t - m) * lax.rsqrt(v + 1e-5) * bng_ref[...] + bnb_ref[...]
  t = jnp.maximum(t, 0.0)
  t = jnp.dot(t, W2_ref[...], preferred_element_type=jnp.float32,
              precision=lax.Precision.DEFAULT) + b2_ref[...]
  m2 = jnp.mean(t, axis=0, keepdims=True)
  v2 = jnp.mean((t - m2) ** 2, axis=0, keepdims=True)
  t = (t - m2) * lax.rsqrt(v2 + 1e-5) * obng_ref[...] + obnb_ref[...]
  return jnp.maximum(t, 0.0)


def _conv_tc(x, parts, W1, b1, bng, bnb, W2, b2, obng, obnb):
  def body(x_ref, parts_ref, W1_ref, b1_ref, bng_ref, bnb_ref, W2_ref, b2_ref,
           obng_ref, obnb_ref, out_ref):
    u = x_ref[...] + parts_ref[0, :N, :] + parts_ref[1, :N, :]
    out_ref[...] = _mlp_bn(u, W1_ref, b1_ref, bng_ref, bnb_ref, W2_ref,
                           b2_ref, obng_ref, obnb_ref)

  return pl.pallas_call(
      body,
      out_shape=jax.ShapeDtypeStruct((N, D), jnp.float32),
  )(x, parts, W1, b1, bng, bnb, W2, b2, obng, obnb)


def _head_tc(x, h1, parts, W1, b1, bng, bnb, W2, b2, obng, obnb, bidx,
             p0_W, p0_b, p2_W, p2_b, out_W, out_b):
  def body(x_ref, h1_ref, parts_ref, W1_ref, b1_ref, bng_ref, bnb_ref,
           W2_ref, b2_ref, obng_ref, obnb_ref, bidx_ref, p0W_ref, p0b_ref,
           p2W_ref, p2b_ref, outW_ref, outb_ref, out_ref):
    u = h1_ref[...] + parts_ref[0, :N, :] + parts_ref[1, :N, :]
    h2 = _mlp_bn(u, W1_ref, b1_ref, bng_ref, bnb_ref, W2_ref, b2_ref,
                 obng_ref, obnb_ref)
    seg = lax.broadcasted_iota(jnp.int32, (B, N), 0)
    onehot = (seg == bidx_ref[...]).astype(jnp.float32)
    pool0 = jnp.dot(onehot, x_ref[...], preferred_element_type=jnp.float32,
                    precision=lax.Precision.DEFAULT)
    pool2 = jnp.dot(onehot, h2, preferred_element_type=jnp.float32,
                    precision=lax.Precision.DEFAULT)
    oh = (jnp.dot(pool0, p0W_ref[...], preferred_element_type=jnp.float32,
                  precision=lax.Precision.DEFAULT) + p0b_ref[...]
          + jnp.dot(pool2, p2W_ref[...], preferred_element_type=jnp.float32,
                    precision=lax.Precision.DEFAULT) + p2b_ref[...])
    oh = jnp.maximum(oh, 0.0)
    out_ref[...] = jnp.dot(oh, outW_ref[...],
                           preferred_element_type=jnp.float32,
                           precision=lax.Precision.DEFAULT) + outb_ref[...]

  return pl.pallas_call(
      body,
      out_shape=jax.ShapeDtypeStruct((B, O), jnp.float32),
  )(x, h1, parts, W1, b1, bng, bnb, W2, b2, obng, obnb, bidx,
    p0_W, p0_b, p2_W, p2_b, out_W, out_b)


def kernel(x, A, batch_index,
           g0_W1, g0_b1, g0_bng, g0_bnb, g0_W2, g0_b2, g0_obng, g0_obnb,
           g1_W1, g1_b1, g1_bng, g1_bnb, g1_W2, g1_b2, g1_obng, g1_obnb,
           p0_W, p0_b, p2_W, p2_b, out_W, out_b):
  idx3 = A.reshape(2, NW, NCHUNK, CH).transpose(1, 2, 0, 3)
  bidx = batch_index.reshape(1, N)

  r2 = lambda a: a.reshape(1, -1)
  zeros_hbm = jnp.asarray(np.zeros((N_PAD, D), np.float32))

  parts0 = _sc_aggregate(x, idx3, zeros_hbm)
  h1 = _conv_tc(x, parts0, g0_W1, r2(g0_b1), r2(g0_bng), r2(g0_bnb),
                g0_W2, r2(g0_b2), r2(g0_obng), r2(g0_obnb))
  parts1 = _sc_aggregate(h1, idx3, zeros_hbm)
  return _head_tc(x, h1, parts1, g1_W1, r2(g1_b1), r2(g1_bng), r2(g1_bnb),
                  g1_W2, r2(g1_b2), r2(g1_obng), r2(g1_obnb), bidx,
                  p0_W, r2(p0_b), p2_W, r2(p2_b), out_W, r2(out_b))


# speedup vs baseline: 1.0558x; 1.0278x over previous
import functools

import jax
import jax.numpy as jnp
import numpy as np
from jax import lax
from jax.experimental import pallas as pl
from jax.experimental.pallas import tpu as pltpu
from jax.experimental.pallas import tpu_sc as plsc

N = 10000
E = 320000
D = 128
H = 128
MID = 32
O = 64
B = 16

NC = 2
NS = 16
NW = NC * NS
EPT = E // NW
CH = 80
NCHUNK = EPT // CH
BC = 32
N_PAD = 10112
STRIPE = N_PAD // NS


def _sc_aggregate(h, idx3, zeros_hbm):
  mesh = plsc.VectorSubcoreMesh(core_axis_name="c", subcore_axis_name="s")

  @functools.partial(
      pl.kernel,
      out_type=jax.ShapeDtypeStruct((NC, N_PAD, D), jnp.float32),
      mesh=mesh,
      scratch_types=[
          pltpu.VMEM((BC, 2, CH), jnp.int32),
          pltpu.VMEM((CH, D), jnp.float32),
          pltpu.VMEM((CH, D), jnp.float32),
          pltpu.VMEM((CH, D), jnp.float32),
          pltpu.VMEM((CH, D), jnp.float32),
          pltpu.VMEM_SHARED((N_PAD, D), jnp.float32),
          pltpu.SemaphoreType.DMA,
          pltpu.SemaphoreType.DMA,
          pltpu.SemaphoreType.DMA,
          pltpu.SemaphoreType.DMA,
      ],
  )
  def agg_kernel(h_hbm, idx_hbm, z_hbm, out_hbm, idx_v,
                 rows_a, rows_b, rows_c, rows_e, acc_s,
                 sem_a, sem_b, sem_c, sem_e):
    c = lax.axis_index("c")
    s = lax.axis_index("s")
    w = c * NS + s


    def _wait(buf, sem):
      pltpu.make_async_copy(h_hbm.at[pl.ds(0, CH)], buf, sem).wait()

    def _gather(iv, j, buf, sem):
      pltpu.async_copy(h_hbm.at[iv.at[j, 1]], buf, sem)

    def _scat(iv, buf, j):
      pltpu.sync_copy(buf, acc_s.at[iv.at[j, 0]], add=True)

    bufs = ((rows_a, sem_a), (rows_b, sem_b), (rows_c, sem_c),
            (rows_e, sem_e))
    K = len(bufs)
    F = K - 1

    def _run_block(iv, cnt, prologue=True):
      if prologue:
        for t in range(F):
          _gather(iv, t, *bufs[t])
      groups, rem = divmod(cnt - F, K)

      def _group(k, carry):
        j = K * k
        for t in range(K):
          buf, sem = bufs[t]
          _wait(buf, sem)
          _gather(iv, j + t + F, *bufs[(t + F) % K])
          _scat(iv, buf, j + t)
        return carry

      lax.fori_loop(0, groups, _group, 0)
      base = K * groups
      for t in range(rem):
        buf, sem = bufs[t % K]
        _wait(buf, sem)
        _gather(iv, base + t + F, *bufs[(t + F) % K])
        _scat(iv, buf, base + t)
      for t in range(rem, rem + F):
        buf, sem = bufs[t % K]
        _wait(buf, sem)
        _scat(iv, buf, cnt - F + (t - rem))

    blocks = ((0, 31), (31, 31), (62, 31), (93, 32))
    pltpu.sync_copy(idx_hbm.at[w, pl.ds(0, BC)], idx_v)
    for t in range(F):
      _gather(idx_v, t, *bufs[t])
    pltpu.sync_copy(z_hbm, acc_s.at[pl.ds(s * STRIPE, STRIPE)])
    plsc.subcore_barrier()
    for b, (off, cnt) in enumerate(blocks):
      if b:
        pltpu.sync_copy(idx_hbm.at[w, pl.ds(off, BC)], idx_v)
      _run_block(idx_v, cnt, prologue=bool(b))
    plsc.subcore_barrier()

    pltpu.sync_copy(acc_s.at[pl.ds(s * STRIPE, STRIPE)],
                    out_hbm.at[c, pl.ds(s * STRIPE, STRIPE)])

  return agg_kernel(h, idx3, zeros_hbm)


def _mlp_bn(u, W1_ref, b1_ref, bng_ref, bnb_ref, W2_ref, b2_ref, obng_ref,
            obnb_ref):
  t = jnp.dot(u, W1_ref[...], preferred_element_type=jnp.float32,
              precision=lax.Precision.DEFAULT) + b1_ref[...]
  m = jnp.mean(t, axis=0, keepdims=True)
  v = jnp.mean(t * t, axis=0, keepdims=True) - m * m
  t = (t - m) * lax.rsqrt(v + 1e-5) * bng_ref[...] + bnb_ref[...]
  t = jnp.maximum(t, 0.0)
  t = jnp.dot(t, W2_ref[...], preferred_element_type=jnp.float32,
              precision=lax.Precision.DEFAULT) + b2_ref[...]
  m2 = jnp.mean(t, axis=0, keepdims=True)
  v2 = jnp.mean(t * t, axis=0, keepdims=True) - m2 * m2
  t = (t - m2) * lax.rsqrt(v2 + 1e-5) * obng_ref[...] + obnb_ref[...]
  return jnp.maximum(t, 0.0)


def _conv_tc(x, parts, W1, b1, bng, bnb, W2, b2, obng, obnb):
  def body(x_ref, parts_ref, W1_ref, b1_ref, bng_ref, bnb_ref, W2_ref, b2_ref,
           obng_ref, obnb_ref, out_ref):
    u = x_ref[...] + parts_ref[0, :N, :] + parts_ref[1, :N, :]
    out_ref[...] = _mlp_bn(u, W1_ref, b1_ref, bng_ref, bnb_ref, W2_ref,
                           b2_ref, obng_ref, obnb_ref)

  return pl.pallas_call(
      body,
      out_shape=jax.ShapeDtypeStruct((N, D), jnp.float32),
  )(x, parts, W1, b1, bng, bnb, W2, b2, obng, obnb)


def _head_tc(x, h1, parts, W1, b1, bng, bnb, W2, b2, obng, obnb, bidx,
             p0_W, p0_b, p2_W, p2_b, out_W, out_b):
  def body(x_ref, h1_ref, parts_ref, W1_ref, b1_ref, bng_ref, bnb_ref,
           W2_ref, b2_ref, obng_ref, obnb_ref, bidx_ref, p0W_ref, p0b_ref,
           p2W_ref, p2b_ref, outW_ref, outb_ref, out_ref):
    u = h1_ref[...] + parts_ref[0, :N, :] + parts_ref[1, :N, :]
    h2 = _mlp_bn(u, W1_ref, b1_ref, bng_ref, bnb_ref, W2_ref, b2_ref,
                 obng_ref, obnb_ref)
    seg = lax.broadcasted_iota(jnp.int32, (B, N), 0)
    onehot = (seg == bidx_ref[...]).astype(jnp.float32)
    pool0 = jnp.dot(onehot, x_ref[...], preferred_element_type=jnp.float32,
                    precision=lax.Precision.DEFAULT)
    pool2 = jnp.dot(onehot, h2, preferred_element_type=jnp.float32,
                    precision=lax.Precision.DEFAULT)
    oh = (jnp.dot(pool0, p0W_ref[...], preferred_element_type=jnp.float32,
                  precision=lax.Precision.DEFAULT) + p0b_ref[...]
          + jnp.dot(pool2, p2W_ref[...], preferred_element_type=jnp.float32,
                    precision=lax.Precision.DEFAULT) + p2b_ref[...])
    oh = jnp.maximum(oh, 0.0)
    out_ref[...] = jnp.dot(oh, outW_ref[...],
                           preferred_element_type=jnp.float32,
                           precision=lax.Precision.DEFAULT) + outb_ref[...]

  return pl.pallas_call(
      body,
      out_shape=jax.ShapeDtypeStruct((B, O), jnp.float32),
  )(x, h1, parts, W1, b1, bng, bnb, W2, b2, obng, obnb, bidx,
    p0_W, p0_b, p2_W, p2_b, out_W, out_b)


def kernel(x, A, batch_index,
           g0_W1, g0_b1, g0_bng, g0_bnb, g0_W2, g0_b2, g0_obng, g0_obnb,
           g1_W1, g1_b1, g1_bng, g1_bnb, g1_W2, g1_b2, g1_obng, g1_obnb,
           p0_W, p0_b, p2_W, p2_b, out_W, out_b):
  idx3 = A.reshape(2, NW, NCHUNK, CH).transpose(1, 2, 0, 3)
  bidx = batch_index.reshape(1, N)

  r2 = lambda a: a.reshape(1, -1)
  zeros_hbm = jnp.asarray(np.zeros((STRIPE, D), np.float32))

  parts0 = _sc_aggregate(x, idx3, zeros_hbm)
  h1 = _conv_tc(x, parts0, g0_W1, r2(g0_b1), r2(g0_bng), r2(g0_bnb),
                g0_W2, r2(g0_b2), r2(g0_obng), r2(g0_obnb))
  parts1 = _sc_aggregate(h1, idx3, zeros_hbm)
  return _head_tc(x, h1, parts1, g1_W1, r2(g1_b1), r2(g1_bng), r2(g1_bnb),
                  g1_W2, r2(g1_b2), r2(g1_obng), r2(g1_obnb), bidx,
                  p0_W, r2(p0_b), p2_W, r2(p2_b), out_W, r2(out_b))
